# baseline (speedup 1.0000x reference)
import jax
import jax.numpy as jnp
from jax.experimental import pallas as pl


def _add_kernel(x_ref, w_ref, o_ref):
    o_ref[...] = x_ref[...] + w_ref[...]


def kernel(inputs, embed_weight):
    bs, T, D = inputs.shape
    blk_t = 1024
    blk_d = 512
    nt = T // blk_t
    nd = D // blk_d
    x2 = inputs.reshape(bs * T, D)
    out = pl.pallas_call(
        _add_kernel,
        grid=(nt, nd, bs),
        in_specs=[
            pl.BlockSpec((blk_t, blk_d), lambda t, d, b: (b * nt + t, d)),
            pl.BlockSpec((blk_t, blk_d), lambda t, d, b: (t, d)),
        ],
        out_specs=pl.BlockSpec((blk_t, blk_d), lambda t, d, b: (b * nt + t, d)),
        out_shape=jax.ShapeDtypeStruct((bs * T, D), inputs.dtype),
    )(x2, embed_weight)
    return out.reshape(bs, T, D)


# speedup vs baseline: 1.1990x; 1.1990x over previous
import jax
import jax.numpy as jnp
from jax.experimental import pallas as pl


def _add_kernel(x_ref, w_ref, o_ref):
    o_ref[...] = x_ref[...] + w_ref[...]


def kernel(inputs, embed_weight):
    bs, T, D = inputs.shape
    blk = 2048
    nt = T // blk
    x2 = inputs.reshape(bs * T, D)
    out = pl.pallas_call(
        _add_kernel,
        grid=(nt, bs),
        in_specs=[
            pl.BlockSpec((blk, D), lambda t, b: (b * nt + t, 0)),
            pl.BlockSpec((blk, D), lambda t, b: (t, 0)),
        ],
        out_specs=pl.BlockSpec((blk, D), lambda t, b: (b * nt + t, 0)),
        out_shape=jax.ShapeDtypeStruct((bs * T, D), inputs.dtype),
    )(x2, embed_weight)
    return out.reshape(bs, T, D)
